# Initial kernel scaffold; baseline (speedup 1.0000x reference)
import functools

import jax
import jax.numpy as jnp
from jax import lax
from jax.experimental import pallas as pl
from jax.experimental.pallas import tpu as pltpu
from jax.experimental.pallas import tpu_sc as plsc

NC = 2
NS = 16
NW = NC * NS
B = 128


def _make_sc_aggregate(n_pad, n_nodes, chunks, with_deg):
    mesh = plsc.VectorSubcoreMesh(core_axis_name="c", subcore_axis_name="s")
    out_type = [jax.ShapeDtypeStruct((NC, n_nodes, 128), jnp.float32)]
    scratch = [
        pltpu.VMEM((chunks, B), jnp.int32),
        pltpu.VMEM((chunks, B), jnp.int32),
        pltpu.VMEM((B, 128), jnp.float32),
        pltpu.VMEM_SHARED((n_pad, 128), jnp.float32),
    ]
    if with_deg:
        out_type.append(jax.ShapeDtypeStruct((NC, n_nodes, 16), jnp.float32))
        scratch += [
            pltpu.VMEM((B, 16), jnp.float32),
            pltpu.VMEM_SHARED((n_pad, 16), jnp.float32),
        ]

    zrows = n_pad // NS
    orows = n_nodes // NS

    @functools.partial(pl.kernel, mesh=mesh, out_type=out_type,
                       scratch_types=scratch)
    def agg_kernel(h_hbm, src_hbm, dst_hbm, z128_hbm, z16_hbm, ones_hbm,
                   *refs):
        if with_deg:
            (agg_out, deg_out, src_v, dst_v, rows_v, agg_sh, ones_v,
             deg_sh) = refs
        else:
            (agg_out, src_v, dst_v, rows_v, agg_sh) = refs
        cid = lax.axis_index("c")
        sid = lax.axis_index("s")
        wid = cid * NS + sid

        pltpu.sync_copy(src_hbm.at[wid], src_v)
        pltpu.sync_copy(dst_hbm.at[wid], dst_v)
        pltpu.sync_copy(z128_hbm.at[pl.ds(sid * zrows, zrows)],
                        agg_sh.at[pl.ds(sid * zrows, zrows)])
        if with_deg:
            pltpu.sync_copy(ones_hbm, ones_v)
            pltpu.sync_copy(z16_hbm.at[pl.ds(sid * zrows, zrows)],
                            deg_sh.at[pl.ds(sid * zrows, zrows)])
        plsc.subcore_barrier()

        @pl.loop(0, chunks)
        def _(j):
            pltpu.sync_copy(h_hbm.at[src_v.at[j]], rows_v)
            pltpu.sync_copy(rows_v, agg_sh.at[dst_v.at[j]], add=True)
            if with_deg:
                pltpu.sync_copy(ones_v, deg_sh.at[dst_v.at[j]], add=True)

        plsc.subcore_barrier()

        pltpu.sync_copy(agg_sh.at[pl.ds(sid * orows, orows)],
                        agg_out.at[cid, pl.ds(sid * orows, orows)])
        if with_deg:
            pltpu.sync_copy(deg_sh.at[pl.ds(sid * orows, orows)],
                            deg_out.at[cid, pl.ds(sid * orows, orows)])

    return agg_kernel


def _tc_layer(pa, dg, x, wl, wr, b, relu):
    n = x.shape[0]
    bn = 1000
    assert n % bn == 0

    def body(pa_ref, dg_ref, x_ref, wl_ref, wr_ref, b_ref, o_ref):
        agg = pa_ref[0] + pa_ref[1]
        deg = dg_ref[0][:, :1] + dg_ref[1][:, :1]
        mean = agg / jnp.maximum(deg, 1.0)
        acc = jnp.dot(mean, wl_ref[...], preferred_element_type=jnp.float32,
                      precision=lax.Precision.HIGHEST)
        acc = acc + jnp.dot(x_ref[...], wr_ref[...],
                            preferred_element_type=jnp.float32,
                            precision=lax.Precision.HIGHEST)
        acc = acc + b_ref[...]
        if relu:
            acc = jnp.maximum(acc, 0.0)
        o_ref[...] = acc

    return pl.pallas_call(
        body,
        grid=(n // bn,),
        in_specs=[
            pl.BlockSpec((NC, bn, 128), lambda i: (0, i, 0)),
            pl.BlockSpec((NC, bn, 16), lambda i: (0, i, 0)),
            pl.BlockSpec((bn, 128), lambda i: (i, 0)),
            pl.BlockSpec((128, 128), lambda i: (0, 0)),
            pl.BlockSpec((128, 128), lambda i: (0, 0)),
            pl.BlockSpec((1, 128), lambda i: (0, 0)),
        ],
        out_specs=pl.BlockSpec((bn, 128), lambda i: (i, 0)),
        out_shape=jax.ShapeDtypeStruct((n, 128), jnp.float32),
    )(pa, dg, x, wl, wr, b.reshape(1, 128))


def kernel(x, edge_index, W1l, W1r, b1, W2l, W2r, b2):
    n = x.shape[0]
    e = edge_index.shape[1]
    chunks = -(-e // (NW * B))
    e_pad = NW * B * chunks
    n_pad = -(-(n + 1) // NS) * NS

    src = jnp.concatenate(
        [edge_index[0], jnp.zeros((e_pad - e,), jnp.int32)]).reshape(
            NW, chunks, B)
    dst = jnp.concatenate(
        [edge_index[1], jnp.full((e_pad - e,), n, jnp.int32)]).reshape(
            NW, chunks, B)
    z128 = jnp.zeros((n_pad, 128), jnp.float32)
    z16 = jnp.zeros((n_pad, 16), jnp.float32)
    ones = jnp.ones((B, 16), jnp.float32)

    agg_deg = _make_sc_aggregate(n_pad, n, chunks, True)
    agg_only = _make_sc_aggregate(n_pad, n, chunks, False)

    pa1, dg = agg_deg(x, src, dst, z128, z16, ones)
    h = _tc_layer(pa1, dg, x, W1l, W1r, b1, relu=True)
    (pa2,) = agg_only(h, src, dst, z128, z16, ones)
    return _tc_layer(pa2, dg, h, W2l, W2r, b2, relu=False)


# speedup vs baseline: 2.8911x; 2.8911x over previous
import functools

import jax
import jax.numpy as jnp
from jax import lax
from jax.experimental import pallas as pl
from jax.experimental.pallas import tpu as pltpu
from jax.experimental.pallas import tpu_sc as plsc

NC = 2
NS = 16
NW = NC * NS
B = 64

_SC_MESH = dict(
    mesh=plsc.VectorSubcoreMesh(core_axis_name="c", subcore_axis_name="s"))


def _make_sc_aggregate(n_pad, chunks):
    half = chunks // 2
    zrows = n_pad // NS

    @functools.partial(
        pl.kernel, **_SC_MESH,
        out_type=jax.ShapeDtypeStruct((NC, n_pad, 128), jnp.float32),
        scratch_types=[
            pltpu.VMEM((half, B), jnp.int32),
            pltpu.VMEM((half, B), jnp.int32),
            pltpu.VMEM((B, 128), jnp.float32),
            pltpu.VMEM_SHARED((n_pad, 128), jnp.float32),
        ])
    def agg_kernel(h_hbm, src_hbm, dst_hbm, z128_hbm,
                   agg_out, src_v, dst_v, rows_v, agg_sh):
        cid = lax.axis_index("c")
        sid = lax.axis_index("s")
        wid = cid * NS + sid
        base = sid * zrows

        pltpu.sync_copy(z128_hbm.at[pl.ds(base, zrows)],
                        agg_sh.at[pl.ds(base, zrows)])
        plsc.subcore_barrier()

        for p in range(2):
            pltpu.sync_copy(src_hbm.at[wid, pl.ds(p * half, half)], src_v)
            pltpu.sync_copy(dst_hbm.at[wid, pl.ds(p * half, half)], dst_v)

            @pl.loop(0, half)
            def _(j):
                pltpu.sync_copy(h_hbm.at[src_v.at[j]], rows_v)
                pltpu.sync_copy(rows_v, agg_sh.at[dst_v.at[j]], add=True)

        plsc.subcore_barrier()

        pltpu.sync_copy(agg_sh.at[pl.ds(base, zrows)],
                        agg_out.at[cid, pl.ds(base, zrows)])

    return agg_kernel


def _make_sc_degree(n_pad, chunks):
    half = chunks // 2
    zrows = n_pad // NS

    @functools.partial(
        pl.kernel, **_SC_MESH,
        out_type=jax.ShapeDtypeStruct((NC, n_pad, 128), jnp.float32),
        scratch_types=[
            pltpu.VMEM((half, B), jnp.int32),
            pltpu.VMEM((B, 128), jnp.float32),
            pltpu.VMEM_SHARED((n_pad, 128), jnp.float32),
        ])
    def deg_kernel(dst_hbm, ones_hbm, z128_hbm,
                   deg_out, dst_v, ones_v, deg_sh):
        cid = lax.axis_index("c")
        sid = lax.axis_index("s")
        wid = cid * NS + sid
        base = sid * zrows

        pltpu.sync_copy(ones_hbm, ones_v)
        pltpu.sync_copy(z128_hbm.at[pl.ds(base, zrows)],
                        deg_sh.at[pl.ds(base, zrows)])
        plsc.subcore_barrier()

        for p in range(2):
            pltpu.sync_copy(dst_hbm.at[wid, pl.ds(p * half, half)], dst_v)

            @pl.loop(0, half)
            def _(j):
                pltpu.sync_copy(ones_v, deg_sh.at[dst_v.at[j]], add=True)

        plsc.subcore_barrier()
        pltpu.sync_copy(deg_sh.at[pl.ds(base, zrows)],
                        deg_out.at[cid, pl.ds(base, zrows)])

    return deg_kernel


def _tc_layer(pa, dg, x, wl, wr, b, relu):
    n = x.shape[0]
    bn = 1000
    assert n % bn == 0

    def body(pa_ref, dg_ref, x_ref, wl_ref, wr_ref, b_ref, o_ref):
        agg = pa_ref[0] + pa_ref[1]
        deg = dg_ref[0][:, :1] + dg_ref[1][:, :1]
        mean = agg / jnp.maximum(deg, 1.0)
        acc = jnp.dot(mean, wl_ref[...], preferred_element_type=jnp.float32,
                      precision=lax.Precision.HIGHEST)
        acc = acc + jnp.dot(x_ref[...], wr_ref[...],
                            preferred_element_type=jnp.float32,
                            precision=lax.Precision.HIGHEST)
        acc = acc + b_ref[...]
        if relu:
            acc = jnp.maximum(acc, 0.0)
        o_ref[...] = acc

    return pl.pallas_call(
        body,
        grid=(n // bn,),
        in_specs=[
            pl.BlockSpec((NC, bn, 128), lambda i: (0, i, 0)),
            pl.BlockSpec((NC, bn, 128), lambda i: (0, i, 0)),
            pl.BlockSpec((bn, 128), lambda i: (i, 0)),
            pl.BlockSpec((128, 128), lambda i: (0, 0)),
            pl.BlockSpec((128, 128), lambda i: (0, 0)),
            pl.BlockSpec((1, 128), lambda i: (0, 0)),
        ],
        out_specs=pl.BlockSpec((bn, 128), lambda i: (i, 0)),
        out_shape=jax.ShapeDtypeStruct((n, 128), jnp.float32),
    )(pa, dg, x, wl, wr, b.reshape(1, 128))


def kernel(x, edge_index, W1l, W1r, b1, W2l, W2r, b2):
    n = x.shape[0]
    e = edge_index.shape[1]
    chunks = -(-e // (NW * B * 16)) * 16
    e_pad = NW * B * chunks
    n_pad = -(-(n + 1) // (NS * 8)) * (NS * 8)

    src = jnp.concatenate(
        [edge_index[0], jnp.zeros((e_pad - e,), jnp.int32)]).reshape(
            NW, chunks, B)
    dst = jnp.concatenate(
        [edge_index[1], jnp.full((e_pad - e,), n_pad - 1, jnp.int32)]).reshape(
            NW, chunks, B)
    z128 = jnp.zeros((n_pad, 128), jnp.float32)
    ones = jnp.ones((B, 128), jnp.float32)

    sc_agg = _make_sc_aggregate(n_pad, chunks)
    sc_deg = _make_sc_degree(n_pad, chunks)

    dg = sc_deg(dst, ones, z128)
    pa1 = sc_agg(x, src, dst, z128)
    h = _tc_layer(pa1, dg, x, W1l, W1r, b1, relu=True)
    pa2 = sc_agg(h, src, dst, z128)
    return _tc_layer(pa2, dg, h, W2l, W2r, b2, relu=False)


# speedup vs baseline: 3.3303x; 1.1519x over previous
import functools

import jax
import jax.numpy as jnp
from jax import lax
from jax.experimental import pallas as pl
from jax.experimental.pallas import tpu as pltpu
from jax.experimental.pallas import tpu_sc as plsc

NC = 2
NS = 16
NW = NC * NS
B = 64

_SC_MESH = dict(
    mesh=plsc.VectorSubcoreMesh(core_axis_name="c", subcore_axis_name="s"))


def _make_sc_aggregate(n_pad, chunks):
    half = chunks // 2
    zrows = n_pad // NS

    @functools.partial(
        pl.kernel, **_SC_MESH,
        out_type=jax.ShapeDtypeStruct((NC, n_pad, 128), jnp.float32),
        scratch_types=[
            pltpu.VMEM((half, B), jnp.int32),
            pltpu.VMEM((half, B), jnp.int32),
            pltpu.VMEM((B, 128), jnp.float32),
            pltpu.VMEM((B, 128), jnp.float32),
            pltpu.VMEM_SHARED((n_pad, 128), jnp.float32),
            pltpu.SemaphoreType.DMA,
            pltpu.SemaphoreType.DMA,
        ])
    def agg_kernel(h_hbm, src_hbm, dst_hbm, z128_hbm,
                   agg_out, src_v, dst_v, rows0, rows1, agg_sh, g0, g1):
        cid = lax.axis_index("c")
        sid = lax.axis_index("s")
        wid = cid * NS + sid
        base = sid * zrows

        pltpu.sync_copy(z128_hbm.at[pl.ds(base, zrows)],
                        agg_sh.at[pl.ds(base, zrows)])
        plsc.subcore_barrier()

        for p in range(2):
            pltpu.sync_copy(src_hbm.at[wid, pl.ds(p * half, half)], src_v)
            pltpu.sync_copy(dst_hbm.at[wid, pl.ds(p * half, half)], dst_v)
            pltpu.async_copy(h_hbm.at[src_v.at[0]], rows0, g0)
            pltpu.async_copy(h_hbm.at[src_v.at[1]], rows1, g1)

            @pl.loop(0, half, step=2)
            def _(j):
                pltpu.make_async_copy(h_hbm.at[src_v.at[0]], rows0, g0).wait()
                pltpu.sync_copy(rows0, agg_sh.at[dst_v.at[j]], add=True)
                pltpu.async_copy(
                    h_hbm.at[src_v.at[jnp.minimum(j + 2, half - 1)]],
                    rows0, g0)
                pltpu.make_async_copy(h_hbm.at[src_v.at[0]], rows1, g1).wait()
                pltpu.sync_copy(rows1, agg_sh.at[dst_v.at[j + 1]], add=True)
                pltpu.async_copy(
                    h_hbm.at[src_v.at[jnp.minimum(j + 3, half - 1)]],
                    rows1, g1)

            pltpu.make_async_copy(h_hbm.at[src_v.at[0]], rows0, g0).wait()
            pltpu.make_async_copy(h_hbm.at[src_v.at[0]], rows1, g1).wait()

        plsc.subcore_barrier()

        pltpu.sync_copy(agg_sh.at[pl.ds(base, zrows)],
                        agg_out.at[cid, pl.ds(base, zrows)])

    return agg_kernel


def _make_sc_degree(n_pad, chunks):
    half = chunks // 2
    zrows = n_pad // NS

    @functools.partial(
        pl.kernel, **_SC_MESH,
        out_type=jax.ShapeDtypeStruct((NC, n_pad, 128), jnp.float32),
        scratch_types=[
            pltpu.VMEM((half, B), jnp.int32),
            pltpu.VMEM((B, 128), jnp.float32),
            pltpu.VMEM_SHARED((n_pad, 128), jnp.float32),
            pltpu.SemaphoreType.DMA,
        ])
    def deg_kernel(dst_hbm, ones_hbm, z128_hbm,
                   deg_out, dst_v, ones_v, deg_sh, sem):
        cid = lax.axis_index("c")
        sid = lax.axis_index("s")
        wid = cid * NS + sid
        base = sid * zrows

        pltpu.sync_copy(ones_hbm, ones_v)
        pltpu.sync_copy(z128_hbm.at[pl.ds(base, zrows)],
                        deg_sh.at[pl.ds(base, zrows)])
        plsc.subcore_barrier()

        for p in range(2):
            pltpu.sync_copy(dst_hbm.at[wid, pl.ds(p * half, half)], dst_v)

            @pl.loop(0, half, step=8)
            def _(j):
                for k in range(8):
                    pltpu.async_copy(ones_v, deg_sh.at[dst_v.at[j + k]], sem,
                                     add=True)
                for k in range(8):
                    pltpu.make_async_copy(ones_v, deg_sh.at[dst_v.at[j]],
                                          sem).wait()

        plsc.subcore_barrier()
        pltpu.sync_copy(deg_sh.at[pl.ds(base, zrows)],
                        deg_out.at[cid, pl.ds(base, zrows)])

    return deg_kernel


def _tc_layer(pa, dg, x, wl, wr, b, relu):
    n = x.shape[0]
    bn = 1000
    assert n % bn == 0

    def body(pa_ref, dg_ref, x_ref, wl_ref, wr_ref, b_ref, o_ref):
        agg = pa_ref[0] + pa_ref[1]
        deg = dg_ref[0][:, :1] + dg_ref[1][:, :1]
        mean = agg / jnp.maximum(deg, 1.0)
        acc = jnp.dot(mean, wl_ref[...], preferred_element_type=jnp.float32,
                      precision=lax.Precision.HIGHEST)
        acc = acc + jnp.dot(x_ref[...], wr_ref[...],
                            preferred_element_type=jnp.float32,
                            precision=lax.Precision.HIGHEST)
        acc = acc + b_ref[...]
        if relu:
            acc = jnp.maximum(acc, 0.0)
        o_ref[...] = acc

    return pl.pallas_call(
        body,
        grid=(n // bn,),
        in_specs=[
            pl.BlockSpec((NC, bn, 128), lambda i: (0, i, 0)),
            pl.BlockSpec((NC, bn, 128), lambda i: (0, i, 0)),
            pl.BlockSpec((bn, 128), lambda i: (i, 0)),
            pl.BlockSpec((128, 128), lambda i: (0, 0)),
            pl.BlockSpec((128, 128), lambda i: (0, 0)),
            pl.BlockSpec((1, 128), lambda i: (0, 0)),
        ],
        out_specs=pl.BlockSpec((bn, 128), lambda i: (i, 0)),
        out_shape=jax.ShapeDtypeStruct((n, 128), jnp.float32),
    )(pa, dg, x, wl, wr, b.reshape(1, 128))


def kernel(x, edge_index, W1l, W1r, b1, W2l, W2r, b2):
    n = x.shape[0]
    e = edge_index.shape[1]
    chunks = -(-e // (NW * B * 16)) * 16
    e_pad = NW * B * chunks
    n_pad = -(-(n + 1) // (NS * 8)) * (NS * 8)

    src = jnp.concatenate(
        [edge_index[0], jnp.zeros((e_pad - e,), jnp.int32)]).reshape(
            NW, chunks, B)
    dst = jnp.concatenate(
        [edge_index[1], jnp.full((e_pad - e,), n_pad - 1, jnp.int32)]).reshape(
            NW, chunks, B)
    z128 = jnp.zeros((n_pad, 128), jnp.float32)
    ones = jnp.ones((B, 128), jnp.float32)

    sc_agg = _make_sc_aggregate(n_pad, chunks)
    sc_deg = _make_sc_degree(n_pad, chunks)

    dg = sc_deg(dst, ones, z128)
    pa1 = sc_agg(x, src, dst, z128)
    h = _tc_layer(pa1, dg, x, W1l, W1r, b1, relu=True)
    pa2 = sc_agg(h, src, dst, z128)
    return _tc_layer(pa2, dg, h, W2l, W2r, b2, relu=False)
